# baseline (speedup 1.0000x reference)
import jax
import jax.numpy as jnp
from jax.experimental import pallas as pl
from jax.experimental.pallas import tpu as pltpu

_N = 10000
_BM = 200
_NS = _N // _BM


def _s(i):
    return jnp.where(i == 0, 0, jnp.where(i <= _NS, i - 1, i - _NS - 1))


def _even_map(i):
    return ((_s(i) // 2) * 2, 0)


def _odd_map(i):
    return ((_s(i) // 2) * 2 + 1, 0)


def _out_stripe(i):
    return (jnp.where(i > _NS, i - _NS - 1, 0), 0)


def _fused_kernel(x_ref, w1_ref, b1_ref, w2_ref, b2_ref, le_ref, lo_ref,
                  o_ref, s1_ref, s2_ref):
    i = pl.program_id(0)
    par = jax.lax.rem(_s(i), 2)

    @pl.when(i == 0)
    def _():
        s1_ref[...] = jnp.dot(x_ref[...], w1_ref[...],
                              preferred_element_type=jnp.float32)

    def pass1(l_ref):
        h = jnp.dot(l_ref[...], s1_ref[...],
                    preferred_element_type=jnp.float32)
        h = jnp.maximum(h + b1_ref[...], 0.0)
        s2_ref[pl.ds((i - 1) * _BM, _BM), :] = jnp.dot(
            h, w2_ref[...], preferred_element_type=jnp.float32)

    def pass2(l_ref):
        o_ref[...] = jnp.dot(l_ref[...], s2_ref[...],
                             preferred_element_type=jnp.float32) + b2_ref[...]

    @pl.when((i >= 1) & (i <= _NS) & (par == 0))
    def _():
        pass1(le_ref)

    @pl.when((i >= 1) & (i <= _NS) & (par == 1))
    def _():
        pass1(lo_ref)

    @pl.when((i > _NS) & (par == 0))
    def _():
        pass2(le_ref)

    @pl.when((i > _NS) & (par == 1))
    def _():
        pass2(lo_ref)


def kernel(Laplacian, feature, W1, b1, W2, b2):
    n, in_dim = feature.shape
    n_hid = W1.shape[1]
    out_dim = W2.shape[1]
    b1r = b1.reshape(1, n_hid)
    b2r = b2.reshape(1, out_dim)

    return pl.pallas_call(
        _fused_kernel,
        grid=(1 + 2 * _NS,),
        in_specs=[
            pl.BlockSpec((n, in_dim), lambda i: (0, 0)),
            pl.BlockSpec((in_dim, n_hid), lambda i: (0, 0)),
            pl.BlockSpec((1, n_hid), lambda i: (0, 0)),
            pl.BlockSpec((n_hid, out_dim), lambda i: (0, 0)),
            pl.BlockSpec((1, out_dim), lambda i: (0, 0)),
            pl.BlockSpec((_BM, n), _even_map),
            pl.BlockSpec((_BM, n), _odd_map),
        ],
        out_specs=pl.BlockSpec((_BM, out_dim), _out_stripe),
        out_shape=jax.ShapeDtypeStruct((n, out_dim), jnp.float32),
        scratch_shapes=[
            pltpu.VMEM((n, n_hid), jnp.float32),
            pltpu.VMEM((n, out_dim), jnp.float32),
        ],
        compiler_params=pltpu.CompilerParams(
            dimension_semantics=("arbitrary",)),
    )(feature, W1, b1r, W2, b2r, Laplacian, Laplacian)


# speedup vs baseline: 1.4497x; 1.4497x over previous
import jax
import jax.numpy as jnp
from jax.experimental import pallas as pl
from jax.experimental.pallas import tpu as pltpu

_N = 10000
_BM = 400
_NS = _N // _BM


def _fused_kernel(x_ref, w1_ref, b1_ref, w2_ref, b2_ref, l_ref,
                  o_ref, s1_ref, s2_ref):
    i = pl.program_id(0)

    @pl.when(i == 0)
    def _():
        s1_ref[...] = jnp.dot(x_ref[...], w1_ref[...],
                              preferred_element_type=jnp.float32)

    @pl.when(i < _NS)
    def _():
        h = jnp.dot(l_ref[...], s1_ref[...],
                    preferred_element_type=jnp.float32)
        h = jnp.maximum(h + b1_ref[...], 0.0)
        s2_ref[pl.ds(i * _BM, _BM), :] = jnp.dot(
            h, w2_ref[...], preferred_element_type=jnp.float32)

    @pl.when(i >= _NS)
    def _():
        o_ref[...] = jnp.dot(l_ref[...], s2_ref[...],
                             preferred_element_type=jnp.float32) + b2_ref[...]


def _l_stripe(i):
    return (jnp.where(i < _NS, i, i - _NS), 0)


def _out_stripe(i):
    return (jnp.where(i >= _NS, i - _NS, 0), 0)


def kernel(Laplacian, feature, W1, b1, W2, b2):
    n, in_dim = feature.shape
    n_hid = W1.shape[1]
    out_dim = W2.shape[1]
    b1r = b1.reshape(1, n_hid)
    b2r = b2.reshape(1, out_dim)

    return pl.pallas_call(
        _fused_kernel,
        grid=(2 * _NS,),
        in_specs=[
            pl.BlockSpec((n, in_dim), lambda i: (0, 0)),
            pl.BlockSpec((in_dim, n_hid), lambda i: (0, 0)),
            pl.BlockSpec((1, n_hid), lambda i: (0, 0)),
            pl.BlockSpec((n_hid, out_dim), lambda i: (0, 0)),
            pl.BlockSpec((1, out_dim), lambda i: (0, 0)),
            pl.BlockSpec((_BM, n), _l_stripe),
        ],
        out_specs=pl.BlockSpec((_BM, out_dim), _out_stripe),
        out_shape=jax.ShapeDtypeStruct((n, out_dim), jnp.float32),
        scratch_shapes=[
            pltpu.VMEM((n, n_hid), jnp.float32),
            pltpu.VMEM((n, out_dim), jnp.float32),
        ],
        compiler_params=pltpu.CompilerParams(
            dimension_semantics=("arbitrary",)),
    )(feature, W1, b1r, W2, b2r, Laplacian)


# speedup vs baseline: 1.4563x; 1.0045x over previous
import jax
import jax.numpy as jnp
from jax.experimental import pallas as pl
from jax.experimental.pallas import tpu as pltpu

_N = 10000
_BM = 400
_NS = _N // _BM


def _fused_kernel(x_ref, w1_ref, b1_ref, w2_ref, b2_ref, l_ref,
                  o_ref, s1_ref, s2_ref):
    i = pl.program_id(0)

    @pl.when(i == 0)
    def _():
        s1_ref[...] = jnp.dot(x_ref[...], w1_ref[...],
                              preferred_element_type=jnp.float32)

    @pl.when(i < _NS)
    def _():
        h = jnp.dot(l_ref[...], s1_ref[...],
                    preferred_element_type=jnp.float32)
        h = jnp.maximum(h + b1_ref[...], 0.0)
        s2_ref[pl.ds(i * _BM, _BM), :] = jnp.dot(
            h, w2_ref[...], preferred_element_type=jnp.float32)

    @pl.when(i >= _NS)
    def _():
        o_ref[...] = jnp.dot(l_ref[...], s2_ref[...],
                             preferred_element_type=jnp.float32) + b2_ref[...]


def _l_stripe(i):
    return (jnp.where(i < _NS, i, 2 * _NS - 1 - i), 0)


def _out_stripe(i):
    return (jnp.where(i >= _NS, 2 * _NS - 1 - i, 0), 0)


def kernel(Laplacian, feature, W1, b1, W2, b2):
    n, in_dim = feature.shape
    n_hid = W1.shape[1]
    out_dim = W2.shape[1]
    b1r = b1.reshape(1, n_hid)
    b2r = b2.reshape(1, out_dim)

    return pl.pallas_call(
        _fused_kernel,
        grid=(2 * _NS,),
        in_specs=[
            pl.BlockSpec((n, in_dim), lambda i: (0, 0)),
            pl.BlockSpec((in_dim, n_hid), lambda i: (0, 0)),
            pl.BlockSpec((1, n_hid), lambda i: (0, 0)),
            pl.BlockSpec((n_hid, out_dim), lambda i: (0, 0)),
            pl.BlockSpec((1, out_dim), lambda i: (0, 0)),
            pl.BlockSpec((_BM, n), _l_stripe),
        ],
        out_specs=pl.BlockSpec((_BM, out_dim), _out_stripe),
        out_shape=jax.ShapeDtypeStruct((n, out_dim), jnp.float32),
        scratch_shapes=[
            pltpu.VMEM((n, n_hid), jnp.float32),
            pltpu.VMEM((n, out_dim), jnp.float32),
        ],
        compiler_params=pltpu.CompilerParams(
            dimension_semantics=("arbitrary",)),
    )(feature, W1, b1r, W2, b2r, Laplacian)
